# baseline (speedup 1.0000x reference)
import functools

import jax
import jax.numpy as jnp
from jax.experimental import pallas as pl
from jax.experimental.pallas import tpu as pltpu

_BLOCK = 2048
_QUART = _BLOCK // 4


def _bf16_triple(x):
    hi = x.astype(jnp.bfloat16).astype(jnp.float32)
    r = x - hi
    mid = r.astype(jnp.bfloat16).astype(jnp.float32)
    lo = (r - mid).astype(jnp.bfloat16).astype(jnp.float32)
    return hi, mid, lo


def _nn_body(d1_ref, d2_ref, dist_ref, idx_ref, d1aug_sc, min_sc, arg_sc, *,
             nblocks, n2):
    step = pl.program_id(0)
    m = d1_ref.shape[0]

    @pl.when(step == 0)
    def _init():
        d1t = jnp.transpose(d1_ref[...], (1, 0))
        d1aug_sc[0:16, :] = d1t * (-2.0)
        d1aug_sc[16:19, :] = jnp.ones((3, m), d1t.dtype)
        min_sc[...] = jnp.full(min_sc.shape, jnp.inf, min_sc.dtype)
        arg_sc[...] = jnp.zeros(arg_sc.shape, arg_sc.dtype)

    d2 = d2_ref[...]
    d1_aug = d1aug_sc[...]

    rid = jax.lax.broadcasted_iota(jnp.int32, (d2.shape[0], 1), 0)
    valid = (rid + step * _BLOCK) < n2
    d2c = jnp.where(valid, d2, 0.0)
    rowsq = (jnp.sum(d2c * d2c, axis=1, keepdims=True)
             + jnp.where(valid, 0.0, 3.0e30))

    hi, mid, lo = _bf16_triple(rowsq)
    d2_aug = jnp.concatenate([d2c, hi, mid, lo], axis=1)

    rows_f = jax.lax.broadcasted_iota(
        jnp.int32, (_QUART, m), 0).astype(jnp.float32)

    def _quarter(aug):
        sh = jax.lax.dot_general(
            aug, d1_aug, (((1,), (0,)), ((), ())),
            preferred_element_type=jnp.float32)
        bmh = jnp.min(sh, axis=0, keepdims=True)
        bah = jnp.min(jnp.where(sh == bmh, rows_f, jnp.float32(3.0e38)),
                      axis=0, keepdims=True)
        return bmh, bah

    parts = [_quarter(d2_aug[i * _QUART:(i + 1) * _QUART])
             for i in range(_BLOCK // _QUART)]
    bm, ba_loc = parts[0]
    for i, (bmh, bah) in enumerate(parts[1:], start=1):
        takeh = bmh < bm
        bm = jnp.where(takeh, bmh, bm)
        ba_loc = jnp.where(takeh, bah + jnp.float32(i * _QUART), ba_loc)
    ba = ba_loc + (step * _BLOCK).astype(jnp.float32)
    run_min = min_sc[...]
    upd = bm < run_min
    min_sc[...] = jnp.where(upd, bm, run_min)
    arg_sc[...] = jnp.where(upd, ba, arg_sc[...])

    @pl.when(step == nblocks - 1)
    def _fin():
        d1sq = jnp.sum(d1aug_sc[0:16, :] * d1aug_sc[0:16, :],
                       axis=0, keepdims=True) * 0.25
        dist = jnp.sqrt(jnp.maximum(min_sc[...] + d1sq, 0.0))
        dist_ref[...] = jnp.transpose(dist, (1, 0))
        idxq = jax.lax.broadcasted_iota(jnp.int32, (m, 1), 0)
        idx2 = jnp.transpose(arg_sc[...], (1, 0)).astype(jnp.int32)
        idx_ref[...] = jnp.concatenate([idxq, idx2], axis=1)


def kernel(desc1, desc2):
    n1, d = desc1.shape
    n2 = desc2.shape[0]
    nblocks = pl.cdiv(n2, _BLOCK)

    match_dists, matches_idxs = pl.pallas_call(
        functools.partial(_nn_body, nblocks=nblocks, n2=n2),
        grid=(nblocks,),
        in_specs=[
            pl.BlockSpec((n1, d), lambda i: (0, 0)),
            pl.BlockSpec((_BLOCK, d), lambda i: (i, 0)),
        ],
        out_specs=[
            pl.BlockSpec((n1, 1), lambda i: (0, 0)),
            pl.BlockSpec((n1, 2), lambda i: (0, 0)),
        ],
        out_shape=[
            jax.ShapeDtypeStruct((n1, 1), jnp.float32),
            jax.ShapeDtypeStruct((n1, 2), jnp.int32),
        ],
        scratch_shapes=[
            pltpu.VMEM((19, n1), jnp.float32),
            pltpu.VMEM((1, n1), jnp.float32),
            pltpu.VMEM((1, n1), jnp.float32),
        ],
    )(desc1, desc2)

    return (match_dists, matches_idxs)


# speedup vs baseline: 1.1003x; 1.1003x over previous
import functools

import jax
import jax.numpy as jnp
from jax.experimental import pallas as pl
from jax.experimental.pallas import tpu as pltpu

_BLOCK = 4096
_QUART = _BLOCK // 16


def _bf16_triple(x):
    hi = x.astype(jnp.bfloat16).astype(jnp.float32)
    r = x - hi
    mid = r.astype(jnp.bfloat16).astype(jnp.float32)
    lo = (r - mid).astype(jnp.bfloat16).astype(jnp.float32)
    return hi, mid, lo


def _nn_body(d1t_ref, d2_ref, dist_ref, idx_ref, min_sc, arg_sc, *,
             nblocks, n2):
    step = pl.program_id(0)

    @pl.when(step == 0)
    def _init():
        min_sc[...] = jnp.full(min_sc.shape, jnp.inf, min_sc.dtype)
        arg_sc[...] = jnp.zeros(arg_sc.shape, arg_sc.dtype)

    d2 = d2_ref[...]
    d1t = d1t_ref[...]

    rid = jax.lax.broadcasted_iota(jnp.int32, (d2.shape[0], 1), 0)
    valid = (rid + step * _BLOCK) < n2
    d2c = jnp.where(valid, d2, 0.0)
    rowsq = (jnp.sum(d2c * d2c, axis=1, keepdims=True)
             + jnp.where(valid, 0.0, 3.0e30))

    hi, mid, lo = _bf16_triple(rowsq)
    d2_aug = jnp.concatenate([d2c, hi, mid, lo], axis=1)
    ones = jnp.ones((3, d1t.shape[1]), d1t.dtype)
    d1_aug = jnp.concatenate([d1t * (-2.0), ones], axis=0)

    rows_f = jax.lax.broadcasted_iota(
        jnp.int32, (_QUART, d1t.shape[1]), 0).astype(jnp.float32)

    def _quarter(aug):
        sh = jax.lax.dot_general(
            aug, d1_aug, (((1,), (0,)), ((), ())),
            preferred_element_type=jnp.float32)
        bmh = jnp.min(sh, axis=0, keepdims=True)
        bah = jnp.min(jnp.where(sh == bmh, rows_f, jnp.float32(3.0e38)),
                      axis=0, keepdims=True)
        return bmh, bah

    parts = [_quarter(d2_aug[i * _QUART:(i + 1) * _QUART])
             for i in range(_BLOCK // _QUART)]
    bm, ba_loc = parts[0]
    for i, (bmh, bah) in enumerate(parts[1:], start=1):
        takeh = bmh < bm
        bm = jnp.where(takeh, bmh, bm)
        ba_loc = jnp.where(takeh, bah + jnp.float32(i * _QUART), ba_loc)
    ba = ba_loc + (step * _BLOCK).astype(jnp.float32)
    run_min = min_sc[...]
    upd = bm < run_min
    min_sc[...] = jnp.where(upd, bm, run_min)
    arg_sc[...] = jnp.where(upd, ba, arg_sc[...])

    @pl.when(step == nblocks - 1)
    def _fin():
        d1sq = jnp.sum(d1t * d1t, axis=0, keepdims=True)
        dist_ref[...] = jnp.sqrt(jnp.maximum(min_sc[...] + d1sq, 0.0))
        idx_ref[...] = arg_sc[...].astype(jnp.int32)


def kernel(desc1, desc2):
    n1, d = desc1.shape
    n2 = desc2.shape[0]
    nblocks = pl.cdiv(n2, _BLOCK)
    d1t = desc1.T

    dist_row, idx_row = pl.pallas_call(
        functools.partial(_nn_body, nblocks=nblocks, n2=n2),
        grid=(nblocks,),
        in_specs=[
            pl.BlockSpec((d, n1), lambda i: (0, 0)),
            pl.BlockSpec((_BLOCK, d), lambda i: (i, 0)),
        ],
        out_specs=[
            pl.BlockSpec((1, n1), lambda i: (0, 0)),
            pl.BlockSpec((1, n1), lambda i: (0, 0)),
        ],
        out_shape=[
            jax.ShapeDtypeStruct((1, n1), jnp.float32),
            jax.ShapeDtypeStruct((1, n1), jnp.int32),
        ],
        scratch_shapes=[
            pltpu.VMEM((1, n1), jnp.float32),
            pltpu.VMEM((1, n1), jnp.float32),
        ],
    )(d1t, desc2)

    match_dists = dist_row.reshape(n1, 1)
    idxs_in_1 = jnp.arange(n1, dtype=jnp.int32).reshape(n1, 1)
    matches_idxs = jnp.concatenate([idxs_in_1, idx_row.reshape(n1, 1)], axis=1)
    return (match_dists, matches_idxs)


# speedup vs baseline: 1.5933x; 1.4480x over previous
import functools

import jax
import jax.numpy as jnp
from jax.experimental import pallas as pl
from jax.experimental.pallas import tpu as pltpu

_BLOCK = 4096
_QUART = _BLOCK // 16


def _bf16_triple(x):
    hi = x.astype(jnp.bfloat16).astype(jnp.float32)
    r = x - hi
    mid = r.astype(jnp.bfloat16).astype(jnp.float32)
    lo = (r - mid).astype(jnp.bfloat16).astype(jnp.float32)
    return hi, mid, lo


def _nn_body(d1t_ref, d2_ref, dist_ref, idx_ref, min_sc, arg_sc, *,
             nblocks, n2):
    step = pl.program_id(0)

    @pl.when(step == 0)
    def _init():
        min_sc[...] = jnp.full(min_sc.shape, jnp.inf, min_sc.dtype)
        arg_sc[...] = jnp.zeros(arg_sc.shape, arg_sc.dtype)

    d2 = d2_ref[...]
    d1t = d1t_ref[...]

    rid = jax.lax.broadcasted_iota(jnp.int32, (d2.shape[0], 1), 0)
    valid = (rid + step * _BLOCK) < n2
    d2c = jnp.where(valid, d2, 0.0)
    rowsq = (jnp.sum(d2c * d2c, axis=1, keepdims=True)
             + jnp.where(valid, 0.0, 3.0e30))

    hi, mid, lo = _bf16_triple(rowsq)
    d2_aug = jnp.concatenate([d2c, hi, mid, lo], axis=1)
    ones = jnp.ones((3, d1t.shape[1]), d1t.dtype)
    d1_aug = jnp.concatenate([d1t * (-2.0), ones], axis=0)

    m = d1t.shape[1]
    shs = []
    for c in range(_BLOCK // _QUART):
        shs.append(jax.lax.dot_general(
            d2_aug[c * _QUART:(c + 1) * _QUART], d1_aug,
            (((1,), (0,)), ((), ())),
            preferred_element_type=jnp.float32))

    nsl = _QUART // 8
    half = len(shs) // 2

    def _scan(chunks, c0):
        rv = chunks[0][0:8]
        ri = jnp.zeros((8, m), jnp.float32) + jnp.float32(c0 * nsl)
        for ci, sh in enumerate(chunks):
            base = (c0 + ci) * nsl
            for i in range(1 if ci == 0 else 0, nsl):
                sl = sh[8 * i:8 * (i + 1)]
                lt = sl < rv
                rv = jnp.where(lt, sl, rv)
                ri = jnp.where(lt, jnp.float32(base + i), ri)
        return rv, ri

    rv0, ri0 = _scan(shs[:half], 0)
    rv1, ri1 = _scan(shs[half:], half)

    p_iota = jax.lax.broadcasted_iota(
        jnp.int32, (8, m), 0).astype(jnp.float32)
    ab0 = ri0 * 8.0 + p_iota
    ab1 = ri1 * 8.0 + p_iota

    bm = jnp.min(jnp.minimum(rv0, rv1), axis=0, keepdims=True)
    big = jnp.float32(3.0e38)
    ba_loc = jnp.minimum(
        jnp.min(jnp.where(rv0 == bm, ab0, big), axis=0, keepdims=True),
        jnp.min(jnp.where(rv1 == bm, ab1, big), axis=0, keepdims=True))
    ba = ba_loc + (step * _BLOCK).astype(jnp.float32)
    run_min = min_sc[...]
    upd = bm < run_min
    min_sc[...] = jnp.where(upd, bm, run_min)
    arg_sc[...] = jnp.where(upd, ba, arg_sc[...])

    @pl.when(step == nblocks - 1)
    def _fin():
        d1sq = jnp.sum(d1t * d1t, axis=0, keepdims=True)
        dist_ref[...] = jnp.sqrt(jnp.maximum(min_sc[...] + d1sq, 0.0))
        idx_ref[...] = arg_sc[...].astype(jnp.int32)


def kernel(desc1, desc2):
    n1, d = desc1.shape
    n2 = desc2.shape[0]
    nblocks = pl.cdiv(n2, _BLOCK)
    d1t = desc1.T

    dist_row, idx_row = pl.pallas_call(
        functools.partial(_nn_body, nblocks=nblocks, n2=n2),
        grid=(nblocks,),
        in_specs=[
            pl.BlockSpec((d, n1), lambda i: (0, 0)),
            pl.BlockSpec((_BLOCK, d), lambda i: (i, 0)),
        ],
        out_specs=[
            pl.BlockSpec((1, n1), lambda i: (0, 0)),
            pl.BlockSpec((1, n1), lambda i: (0, 0)),
        ],
        out_shape=[
            jax.ShapeDtypeStruct((1, n1), jnp.float32),
            jax.ShapeDtypeStruct((1, n1), jnp.int32),
        ],
        scratch_shapes=[
            pltpu.VMEM((1, n1), jnp.float32),
            pltpu.VMEM((1, n1), jnp.float32),
        ],
    )(d1t, desc2)

    match_dists = dist_row.reshape(n1, 1)
    idxs_in_1 = jnp.arange(n1, dtype=jnp.int32).reshape(n1, 1)
    matches_idxs = jnp.concatenate([idxs_in_1, idx_row.reshape(n1, 1)], axis=1)
    return (match_dists, matches_idxs)
